# baseline (speedup 1.0000x reference)
import functools

import jax
import jax.numpy as jnp
from jax.experimental import pallas as pl
from jax.experimental.pallas import tpu as pltpu

MARGIN = 1.0
EPS = 1e-6
BLK = 1024
AUG = 136
NCLS = 256


def _loss_kernel(ij_ref, emb_ref, trow_ref, tcol_ref, out_ref,
                 a16_ref, b16_ref, w_ref, p_ref, q_ref, oh_ref, eye_ref,
                 z_ref, mn_ref, thr_ref, *, nsteps, dim, b):
    t = pl.program_id(0)

    @pl.when(t == 0)
    def _init():
        x = emb_ref[...]
        sq = jnp.sum(x * x, axis=1, keepdims=True)
        s = jnp.sum(x, axis=1, keepdims=True)
        half_k = 0.5 * dim * EPS * EPS
        p = sq + (2.0 * EPS) * s + half_k
        q = sq - (2.0 * EPS) * s + half_k
        one = jnp.ones((b, 1), jnp.float32)
        zero = jnp.zeros((b, 1), jnp.float32)
        pad = jnp.zeros((b, AUG - dim - 2), jnp.float32)
        a16_ref[...] = jnp.concatenate(
            [-2.0 * x, zero, one, pad], axis=1).astype(jnp.bfloat16)
        b16_ref[...] = jnp.concatenate(
            [x, zero, q, pad], axis=1).astype(jnp.bfloat16)
        w_ref[...] = jnp.concatenate([x, sq, one, pad], axis=1)
        p_ref[...] = p
        q_ref[...] = q
        rmax2 = jnp.max(sq)
        qmax = jnp.max(jnp.abs(q))
        thr_ref[0, 0] = (MARGIN * MARGIN
                         + 2.0 ** -8 * (4.1 * rmax2 + 2.1 * qmax) + 1e-6)
        rr = jax.lax.broadcasted_iota(jnp.int32, (BLK, BLK), 0)
        cc = jax.lax.broadcasted_iota(jnp.int32, (BLK, BLK), 1)
        eye_ref[...] = jnp.where(rr == cc, jnp.float32(3.0e38), 0.0)
        cls = jax.lax.broadcasted_iota(jnp.int32, (NCLS, b), 0)
        tall = jnp.broadcast_to(tcol_ref[...], (NCLS, b))
        oh_ref[...] = jnp.where(cls == tall, 1.0, 0.0)
        a_seg = jax.lax.dot_general(
            oh_ref[...], w_ref[...], (((1,), (0,)), ((), ())),
            preferred_element_type=jnp.float32)
        s_c = a_seg[:, 0:dim]
        sq_c = a_seg[:, dim:dim + 1]
        n_c = a_seg[:, dim + 1:dim + 2]
        pos = (2.0 * jnp.sum(n_c * sq_c) - 2.0 * jnp.sum(s_c * s_c)
               + (dim * EPS * EPS) * jnp.sum(n_c * n_c))
        out_ref[...] = pos.reshape(1, 1)

    @pl.when(t < nsteps)
    def _matmul():
        ri = ij_ref[0, t] * BLK
        rj = ij_ref[1, t] * BLK
        zq = jax.lax.dot_general(
            a16_ref[pl.ds(ri, BLK), :], b16_ref[pl.ds(rj, BLK), :],
            (((1,), (1,)), ((), ())), preferred_element_type=jnp.float32
        )
        z_ref[pl.ds(t % 2, 1), :, :] = zq.reshape(1, BLK, BLK)

    @pl.when(t > 0)
    def _reduce():
        tp = t - 1
        rip = ij_ref[0, tp] * BLK
        is_diag = ij_ref[0, tp] == ij_ref[1, tp]
        zp = z_ref[pl.ds((t + 1) % 2, 1), :, :].reshape(BLK, BLK)
        pcol = p_ref[pl.ds(rip, BLK), :]

        @pl.when(is_diag)
        def _min_diag():
            zm = zp + eye_ref[...]
            mn_ref[0, 0] = jnp.min(pcol + jnp.min(zm, axis=1, keepdims=True))

        @pl.when(jnp.logical_not(is_diag))
        def _min_off():
            mn_ref[0, 0] = jnp.min(pcol + jnp.min(zp, axis=1, keepdims=True))

        @pl.when(mn_ref[0, 0] < thr_ref[0, 0])
        def _hinge():
            rjp = ij_ref[1, tp] * BLK
            xi = emb_ref[pl.ds(rip, BLK), :]
            xj = emb_ref[pl.ds(rjp, BLK), :]
            one = jnp.ones((BLK, 1), jnp.float32)
            pad = jnp.zeros((BLK, AUG - dim - 2), jnp.float32)
            ua = jnp.concatenate([-2.0 * xi, pcol, one, pad], axis=1)
            vb = jnp.concatenate(
                [xj, one, q_ref[pl.ds(rjp, BLK), :], pad], axis=1)
            z = jax.lax.dot_general(
                ua, vb, (((1,), (1,)), ((), ())),
                preferred_element_type=jnp.float32)
            d = jnp.sqrt(jnp.maximum(z, 0.0))
            h = jnp.maximum(MARGIN - d, 0.0)
            same = (trow_ref[pl.ds(rip, BLK), :]
                    == tcol_ref[:, pl.ds(rjp, BLK)])
            hs = jnp.where(same, 0.0, h * h)
            wgt = jnp.where(is_diag, 1.0, 2.0)
            out_ref[...] += (wgt * jnp.sum(hs)).reshape(1, 1)

    @pl.when(t == nsteps)
    def _finalize():
        out_ref[...] = 0.5 * (out_ref[...] - b * dim * EPS * EPS)


def kernel(embeddings, target):
    b, dim = embeddings.shape
    nblk = b // BLK
    pairs = [(i, j) for i in range(nblk) for j in range(nblk) if j >= i]
    nsteps = len(pairs)
    ij = jnp.asarray(pairs + [(0, 0)], dtype=jnp.int32).T
    t_row = target.reshape(b, 1)
    t_col = target.reshape(1, b)
    grid_spec = pltpu.PrefetchScalarGridSpec(
        num_scalar_prefetch=1,
        grid=(nsteps + 1,),
        in_specs=[
            pl.BlockSpec((b, dim), lambda t, ij: (0, 0)),
            pl.BlockSpec((b, 1), lambda t, ij: (0, 0)),
            pl.BlockSpec((1, b), lambda t, ij: (0, 0)),
        ],
        out_specs=pl.BlockSpec((1, 1), lambda t, ij: (0, 0)),
        scratch_shapes=[
            pltpu.VMEM((b, AUG), jnp.bfloat16),
            pltpu.VMEM((b, AUG), jnp.bfloat16),
            pltpu.VMEM((b, AUG), jnp.float32),
            pltpu.VMEM((b, 1), jnp.float32),
            pltpu.VMEM((b, 1), jnp.float32),
            pltpu.VMEM((NCLS, b), jnp.float32),
            pltpu.VMEM((BLK, BLK), jnp.float32),
            pltpu.VMEM((2, BLK, BLK), jnp.float32),
            pltpu.SMEM((1, 1), jnp.float32),
            pltpu.SMEM((1, 1), jnp.float32),
        ],
    )
    out = pl.pallas_call(
        functools.partial(_loss_kernel, nsteps=nsteps, dim=dim, b=b),
        grid_spec=grid_spec,
        out_shape=jax.ShapeDtypeStruct((1, 1), jnp.float32),
    )(ij, embeddings, t_row, t_col)
    return out[0, 0]
